# baseline (speedup 1.0000x reference)
import functools

import jax
import jax.numpy as jnp
from jax import lax
from jax.experimental import pallas as pl
from jax.experimental.pallas import tpu as pltpu
from jax.experimental.pallas import tpu_sc as plsc

_F, _N, _B, _C = 2, 2, 16384, 1000
_M = 0.3
_T = 0.1
_LAMBDA_P = 0.1
_RHO = 0.01

_NS = _F * _N
_ROWS = _NS * _B
_NC, _NSUB, _L = 2, 16, 16
_NW = _NC * _NSUB
_RPW = _ROWS // _NW
_WPS = _B // _RPW
_CH = 16
_NCHUNK = _RPW // _CH
_NEG = -1e10


def _row_max(buf, r):
    def col_body(k, m):
        return jnp.maximum(m, buf[r, pl.ds(k * _L, _L)])
    m = buf[r, pl.ds(0, _L)]
    m = lax.fori_loop(1, _C // _L, col_body, m)
    m = jnp.maximum(m, buf[r, pl.ds(_C - _L, _L)])
    return jnp.max(m)


def _process_chunk(buf, lab_v, i, acc1, acc2):
    rows = lax.broadcasted_iota(jnp.int32, (_L,), 0)
    labs = lab_v[pl.ds(i * _CH, _CH)]
    fy = plsc.load_gather(buf, [rows, labs])
    plsc.store_scatter(buf, [rows, labs], jnp.full((_L,), _NEG, jnp.float32))
    acc1 = acc1 + jnp.maximum(_M + _T - fy, 0.0)

    def row_body(r, a):
        return a + jnp.maximum(_M + _row_max(buf, r), 0.0)

    acc2 = lax.fori_loop(0, _CH, row_body, acc2)
    return acc1, acc2


def _sc_partials(y, label):
    mesh = plsc.VectorSubcoreMesh(core_axis_name="c", subcore_axis_name="s")

    @functools.partial(
        pl.kernel,
        mesh=mesh,
        out_type=jax.ShapeDtypeStruct((_NW, _L), jnp.float32),
        scratch_types=[
            pltpu.VMEM((_RPW,), jnp.int32),
            pltpu.VMEM((_CH, _C), jnp.float32),
            pltpu.VMEM((_CH, _C), jnp.float32),
            pltpu.VMEM((_L,), jnp.float32),
            pltpu.SemaphoreType.DMA,
            pltpu.SemaphoreType.DMA,
        ],
        compiler_params=pltpu.CompilerParams(needs_layout_passes=False),
    )
    def sc_k(y_hbm, lab_hbm, out_hbm, lab_v, buf0, buf1, part_v, sem0, sem1):
        wid = lax.axis_index("s") * _NC + lax.axis_index("c")
        sl = wid // _WPS
        f = sl // _N
        n = sl % _N
        b0 = (wid % _WPS) * _RPW

        pltpu.sync_copy(lab_hbm.at[pl.ds(b0, _RPW)], lab_v)

        def src(i):
            return y_hbm.at[f, n, pl.ds(b0 + i * _CH, _CH)]

        pltpu.make_async_copy(src(0), buf0, sem0).start()

        def step(j, carry):
            acc1, acc2 = carry
            pltpu.make_async_copy(src(2 * j + 1), buf1, sem1).start()
            pltpu.make_async_copy(src(2 * j), buf0, sem0).wait()
            acc1, acc2 = _process_chunk(buf0, lab_v, 2 * j, acc1, acc2)

            @pl.when(j < _NCHUNK // 2 - 1)
            def _():
                pltpu.make_async_copy(src(2 * j + 2), buf0, sem0).start()

            pltpu.make_async_copy(src(2 * j + 1), buf1, sem1).wait()
            acc1, acc2 = _process_chunk(buf1, lab_v, 2 * j + 1, acc1, acc2)
            return acc1, acc2

        acc1 = jnp.zeros((_L,), jnp.float32)
        acc1, acc2 = lax.fori_loop(
            0, _NCHUNK // 2, step, (acc1, jnp.float32(0.0))
        )

        lane0 = lax.broadcasted_iota(jnp.int32, (_L,), 0) == 0
        part_v[...] = acc1 + jnp.where(lane0, acc2, 0.0)
        pltpu.sync_copy(part_v, out_hbm.at[wid])

    return sc_k(y, label)


def _combine_body(p_ref, pc_ref, out_ref):
    pc = pc_ref[0, 0]
    total = jnp.sum(p_ref[...]) * (1.0 / (_NS * _B))
    out_ref[0, 0] = total + _LAMBDA_P * pc + (_RHO / 2.0) * pc * pc


def kernel(y, label, power_ratio, power_consumption):
    del power_ratio
    partials = _sc_partials(y, label)
    pc = power_consumption.reshape(1, 1)
    out = pl.pallas_call(
        _combine_body,
        in_specs=[
            pl.BlockSpec((_NW, _L), lambda: (0, 0)),
            pl.BlockSpec(memory_space=pltpu.SMEM),
        ],
        out_specs=pl.BlockSpec(memory_space=pltpu.SMEM),
        out_shape=jax.ShapeDtypeStruct((1, 1), jnp.float32),
    )(partials, pc)
    return out.reshape(1)


# speedup vs baseline: 2.2875x; 2.2875x over previous
import functools

import jax
import jax.numpy as jnp
from jax import lax
from jax.experimental import pallas as pl
from jax.experimental.pallas import tpu as pltpu
from jax.experimental.pallas import tpu_sc as plsc

_F, _N, _B, _C = 2, 2, 16384, 1000
_M = 0.3
_T = 0.1
_LAMBDA_P = 0.1
_RHO = 0.01

_NS = _F * _N
_ROWS = _NS * _B
_NC, _NSUB, _L = 2, 16, 16
_NW = _NC * _NSUB
_RPW = _ROWS // _NW
_WPS = _B // _RPW
_CH = 16
_NCHUNK = _RPW // _CH
_NEG = -1e10


def _row_max(buf, r):
    acc = [buf[r, pl.ds(k * _L, _L)] for k in range(4)]
    for k in range(4, _C // _L):
        acc[k % 4] = jnp.maximum(acc[k % 4], buf[r, pl.ds(k * _L, _L)])
    acc[0] = jnp.maximum(acc[0], buf[r, pl.ds(_C - _L, _L)])
    m = jnp.maximum(jnp.maximum(acc[0], acc[1]), jnp.maximum(acc[2], acc[3]))
    return jnp.max(m)


def _process_chunk(buf, lab_v, i, acc1, acc2):
    rows = lax.broadcasted_iota(jnp.int32, (_L,), 0)
    labs = lab_v[pl.ds(i * _CH, _CH)]
    fy = plsc.load_gather(buf, [rows, labs])
    plsc.store_scatter(buf, [rows, labs], jnp.full((_L,), _NEG, jnp.float32))
    acc1 = acc1 + jnp.maximum(_M + _T - fy, 0.0)

    def row_body(r, a):
        return a + jnp.maximum(_M + _row_max(buf, r), 0.0)

    acc2 = lax.fori_loop(0, _CH, row_body, acc2)
    return acc1, acc2


def _sc_partials(y, label):
    mesh = plsc.VectorSubcoreMesh(core_axis_name="c", subcore_axis_name="s")

    @functools.partial(
        pl.kernel,
        mesh=mesh,
        out_type=jax.ShapeDtypeStruct((_NW, _L), jnp.float32),
        scratch_types=[
            pltpu.VMEM((_RPW,), jnp.int32),
            pltpu.VMEM((_CH, _C), jnp.float32),
            pltpu.VMEM((_CH, _C), jnp.float32),
            pltpu.VMEM((_L,), jnp.float32),
            pltpu.SemaphoreType.DMA,
            pltpu.SemaphoreType.DMA,
        ],
        compiler_params=pltpu.CompilerParams(needs_layout_passes=False),
    )
    def sc_k(y_hbm, lab_hbm, out_hbm, lab_v, buf0, buf1, part_v, sem0, sem1):
        wid = lax.axis_index("s") * _NC + lax.axis_index("c")
        sl = wid // _WPS
        f = sl // _N
        n = sl % _N
        b0 = (wid % _WPS) * _RPW

        pltpu.sync_copy(lab_hbm.at[pl.ds(b0, _RPW)], lab_v)

        def src(i):
            return y_hbm.at[f, n, pl.ds(b0 + i * _CH, _CH)]

        pltpu.make_async_copy(src(0), buf0, sem0).start()

        def step(j, carry):
            acc1, acc2 = carry
            pltpu.make_async_copy(src(2 * j + 1), buf1, sem1).start()
            pltpu.make_async_copy(src(2 * j), buf0, sem0).wait()
            acc1, acc2 = _process_chunk(buf0, lab_v, 2 * j, acc1, acc2)

            @pl.when(j < _NCHUNK // 2 - 1)
            def _():
                pltpu.make_async_copy(src(2 * j + 2), buf0, sem0).start()

            pltpu.make_async_copy(src(2 * j + 1), buf1, sem1).wait()
            acc1, acc2 = _process_chunk(buf1, lab_v, 2 * j + 1, acc1, acc2)
            return acc1, acc2

        acc1 = jnp.zeros((_L,), jnp.float32)
        acc1, acc2 = lax.fori_loop(
            0, _NCHUNK // 2, step, (acc1, jnp.float32(0.0))
        )

        lane0 = lax.broadcasted_iota(jnp.int32, (_L,), 0) == 0
        part_v[...] = acc1 + jnp.where(lane0, acc2, 0.0)
        pltpu.sync_copy(part_v, out_hbm.at[wid])

    return sc_k(y, label)


def _combine_body(p_ref, pc_ref, out_ref):
    pc = pc_ref[0, 0]
    total = jnp.sum(p_ref[...]) * (1.0 / (_NS * _B))
    out_ref[0, 0] = total + _LAMBDA_P * pc + (_RHO / 2.0) * pc * pc


def kernel(y, label, power_ratio, power_consumption):
    del power_ratio
    partials = _sc_partials(y, label)
    pc = power_consumption.reshape(1, 1)
    out = pl.pallas_call(
        _combine_body,
        in_specs=[
            pl.BlockSpec((_NW, _L), lambda: (0, 0)),
            pl.BlockSpec(memory_space=pltpu.SMEM),
        ],
        out_specs=pl.BlockSpec(memory_space=pltpu.SMEM),
        out_shape=jax.ShapeDtypeStruct((1, 1), jnp.float32),
    )(partials, pc)
    return out.reshape(1)


# speedup vs baseline: 6.4390x; 2.8148x over previous
import jax
import jax.numpy as jnp
from jax.experimental import pallas as pl
from jax.experimental.pallas import tpu as pltpu

_F, _N, _B, _C = 2, 2, 16384, 1000
_M = 0.3
_T = 0.1
_LAMBDA_P = 0.1
_RHO = 0.01

_CM = 40
_NJ = _C // _CM
_NS = _F * _N
_NEG = -1e10


def _loss_body(y_ref, lab_ref, pc_ref, out_ref, fy_scr, mx_scr):
    s = pl.program_id(0)
    j = pl.program_id(1)

    @pl.when((s == 0) & (j == 0))
    def _init():
        out_ref[0, 0] = 0.0

    @pl.when(j == 0)
    def _reset():
        fy_scr[...] = jnp.zeros((1, _B), jnp.float32)
        mx_scr[...] = jnp.full((1, _B), _NEG, jnp.float32)

    yb = y_ref[0, 0]
    lab = lab_ref[...]
    crow = jax.lax.broadcasted_iota(jnp.int32, (_CM, _B), 0) + j * _CM
    mask = crow == lab
    fy_scr[...] += jnp.sum(jnp.where(mask, yb, 0.0), axis=0, keepdims=True)
    blk_mx = jnp.max(jnp.where(mask, _NEG, yb), axis=0, keepdims=True)
    mx_scr[...] = jnp.maximum(mx_scr[...], blk_mx)

    @pl.when(j == _NJ - 1)
    def _slice_done():
        l = jnp.maximum(_M + _T - fy_scr[...], 0.0) + jnp.maximum(
            _M + mx_scr[...], 0.0
        )
        out_ref[0, 0] += jnp.sum(l) * (1.0 / (_NS * _B))

    @pl.when((s == _NS - 1) & (j == _NJ - 1))
    def _fini():
        pc = pc_ref[0, 0]
        out_ref[0, 0] += _LAMBDA_P * pc + (_RHO / 2.0) * pc * pc


def kernel(y, label, power_ratio, power_consumption):
    del power_ratio
    yt = jnp.transpose(y, (0, 1, 3, 2))
    lab2 = label[None, :]
    pc = power_consumption.reshape(1, 1)

    out = pl.pallas_call(
        _loss_body,
        grid=(_NS, _NJ),
        in_specs=[
            pl.BlockSpec((1, 1, _CM, _B), lambda s, j: (s // _N, s % _N, j, 0)),
            pl.BlockSpec((1, _B), lambda s, j: (0, 0)),
            pl.BlockSpec(memory_space=pltpu.SMEM),
        ],
        out_specs=pl.BlockSpec(memory_space=pltpu.SMEM),
        out_shape=jax.ShapeDtypeStruct((1, 1), jnp.float32),
        scratch_shapes=[
            pltpu.VMEM((1, _B), jnp.float32),
            pltpu.VMEM((1, _B), jnp.float32),
        ],
        compiler_params=pltpu.CompilerParams(
            dimension_semantics=("arbitrary", "arbitrary"),
        ),
    )(yt, lab2, pc)
    return out.reshape(1)


# speedup vs baseline: 10.2566x; 1.5929x over previous
import jax
import jax.numpy as jnp
from jax.experimental import pallas as pl
from jax.experimental.pallas import tpu as pltpu

_F, _N, _B, _C = 2, 2, 16384, 1000
_M = 0.3
_T = 0.1
_LAMBDA_P = 0.1
_RHO = 0.01

_CM = 200
_NJ = _C // _CM
_NS = _F * _N
_NEG = -1e10


def _loss_body(y_ref, lab_ref, pc_ref, out_ref, fy_scr, mx_scr):
    s = pl.program_id(0)
    j = pl.program_id(1)

    @pl.when((s == 0) & (j == 0))
    def _init():
        out_ref[0, 0] = 0.0

    @pl.when(j == 0)
    def _reset():
        fy_scr[...] = jnp.zeros((1, _B), jnp.float32)
        mx_scr[...] = jnp.full((1, _B), _NEG, jnp.float32)

    yb = y_ref[0, 0]
    lab = lab_ref[...]
    crow = jax.lax.broadcasted_iota(jnp.int32, (_CM, _B), 0) + j * _CM
    mask = crow == lab
    fy_scr[...] += jnp.sum(jnp.where(mask, yb, 0.0), axis=0, keepdims=True)
    blk_mx = jnp.max(jnp.where(mask, _NEG, yb), axis=0, keepdims=True)
    mx_scr[...] = jnp.maximum(mx_scr[...], blk_mx)

    @pl.when(j == _NJ - 1)
    def _slice_done():
        l = jnp.maximum(_M + _T - fy_scr[...], 0.0) + jnp.maximum(
            _M + mx_scr[...], 0.0
        )
        out_ref[0, 0] += jnp.sum(l) * (1.0 / (_NS * _B))

    @pl.when((s == _NS - 1) & (j == _NJ - 1))
    def _fini():
        pc = pc_ref[0, 0]
        out_ref[0, 0] += _LAMBDA_P * pc + (_RHO / 2.0) * pc * pc


def kernel(y, label, power_ratio, power_consumption):
    del power_ratio
    yt = jnp.transpose(y, (0, 1, 3, 2))
    lab2 = label[None, :]
    pc = power_consumption.reshape(1, 1)

    out = pl.pallas_call(
        _loss_body,
        grid=(_NS, _NJ),
        in_specs=[
            pl.BlockSpec((1, 1, _CM, _B), lambda s, j: (s // _N, s % _N, j, 0)),
            pl.BlockSpec((1, _B), lambda s, j: (0, 0)),
            pl.BlockSpec(memory_space=pltpu.SMEM),
        ],
        out_specs=pl.BlockSpec(memory_space=pltpu.SMEM),
        out_shape=jax.ShapeDtypeStruct((1, 1), jnp.float32),
        scratch_shapes=[
            pltpu.VMEM((1, _B), jnp.float32),
            pltpu.VMEM((1, _B), jnp.float32),
        ],
        compiler_params=pltpu.CompilerParams(
            dimension_semantics=("arbitrary", "arbitrary"),
        ),
    )(yt, lab2, pc)
    return out.reshape(1)
